# Initial kernel scaffold; baseline (speedup 1.0000x reference)
import jax
import jax.numpy as jnp
from jax.experimental import pallas as pl


def kernel(x, edge_index, enc_w, enc_b, dec_w, dec_b, conv_lw, conv_lb, conv_rw, gg_lw, gg_lb, gg_rw, q_w, q_b):
    raise NotImplementedError("write your pallas kernel here")


# speedup vs baseline: 3.7185x; 3.7185x over previous
import functools

import jax
import jax.numpy as jnp
from jax import lax
from jax.experimental import pallas as pl
from jax.experimental.pallas import tpu as pltpu
from jax.experimental.pallas import tpu_sc as plsc

NN = 10000
NP = 10240
EE = 320000
F = 128
NCLS = 40
NC = 2
NT = 16
W = 80
RPT = NP // NT
CCH = NP // NT


def _mesh():
    return plsc.VectorSubcoreMesh(core_axis_name="c", subcore_axis_name="s")


def _make_edge_pass(split_edges: bool):
    epw = EE // (NC * NT) if split_edges else EE // NT
    nwin = epw // W
    SB = RPT // 5

    @functools.partial(
        pl.kernel,
        out_type=(jax.ShapeDtypeStruct((NP, F), jnp.float32),
                  jax.ShapeDtypeStruct((NP, F), jnp.float32)),
        mesh=_mesh(),
        scratch_types=[
            pltpu.VMEM((W,), jnp.int32),
            pltpu.VMEM((W,), jnp.int32),
            pltpu.VMEM((W, F), jnp.float32),
            pltpu.VMEM((SB, F), jnp.float32),
            pltpu.VMEM_SHARED((NP, F), jnp.float32),
        ],
    )
    def ep(t0, t1, gidx, sidx, o0, o1, gi_v, si_v, rows_v, stage_v, acc):
        c = lax.axis_index("c")
        s = lax.axis_index("s")

        def zrow(i, carry):
            for j in range(F // 16):
                stage_v[i, pl.ds(j * 16, 16)] = jnp.zeros((16,), jnp.float32)
            return carry

        lax.fori_loop(0, SB, zrow, 0)
        for j in range(RPT // SB):
            pltpu.sync_copy(stage_v, acc.at[pl.ds(s * RPT + j * SB, SB)])
        plsc.subcore_barrier()

        if split_edges:
            ebase = (c * NT + s) * epw
        else:
            ebase = s * epw

        def body(i, carry):
            e0 = ebase + i * W
            pltpu.sync_copy(gidx.at[pl.ds(e0, W)], gi_v)
            pltpu.sync_copy(sidx.at[pl.ds(e0, W)], si_v)

            @pl.when(c == 0)
            def _():
                pltpu.sync_copy(t0.at[gi_v], rows_v)

            @pl.when(c == 1)
            def _():
                pltpu.sync_copy(t1.at[gi_v], rows_v)

            pltpu.sync_copy(rows_v, acc.at[si_v], add=True)
            return carry

        lax.fori_loop(0, nwin, body, 0)
        plsc.subcore_barrier()

        for j in range(RPT // SB):
            r0 = s * RPT + j * SB
            pltpu.sync_copy(acc.at[pl.ds(r0, SB)], stage_v)

            @pl.when(c == 0)
            def _():
                pltpu.sync_copy(stage_v, o0.at[pl.ds(r0, SB)])

            @pl.when(c == 1)
            def _():
                pltpu.sync_copy(stage_v, o1.at[pl.ds(r0, SB)])

    return ep


_edge_pass_split = _make_edge_pass(True)
_edge_pass_dual = _make_edge_pass(False)


@functools.partial(
    pl.kernel,
    out_type=(jax.ShapeDtypeStruct((NP,), jnp.float32),
              jax.ShapeDtypeStruct((NP,), jnp.float32)),
    mesh=_mesh(),
    scratch_types=[
        pltpu.VMEM((W,), jnp.int32),
        pltpu.VMEM((W,), jnp.float32),
        pltpu.VMEM((CCH,), jnp.float32),
        pltpu.VMEM_SHARED((NP,), jnp.float32),
    ],
)
def _counts(src, dst, o_cs, o_cd, idx_v, ones_v, stage_v, acc):
    c = lax.axis_index("c")
    s = lax.axis_index("s")
    for j in range(W // 16):
        ones_v[pl.ds(j * 16, 16)] = jnp.ones((16,), jnp.float32)

    def zchunk(i, carry):
        stage_v[pl.ds(i * 16, 16)] = jnp.zeros((16,), jnp.float32)
        return carry

    lax.fori_loop(0, CCH // 16, zchunk, 0)
    pltpu.sync_copy(stage_v, acc.at[pl.ds(s * CCH, CCH)])
    plsc.subcore_barrier()

    epw = EE // NT

    def body(i, carry):
        e0 = s * epw + i * W

        @pl.when(c == 0)
        def _():
            pltpu.sync_copy(src.at[pl.ds(e0, W)], idx_v)

        @pl.when(c == 1)
        def _():
            pltpu.sync_copy(dst.at[pl.ds(e0, W)], idx_v)

        pltpu.sync_copy(ones_v, acc.at[idx_v], add=True)
        return carry

    lax.fori_loop(0, epw // W, body, 0)
    plsc.subcore_barrier()

    pltpu.sync_copy(acc.at[pl.ds(s * CCH, CCH)], stage_v)

    @pl.when(c == 0)
    def _():
        pltpu.sync_copy(stage_v, o_cs.at[pl.ds(s * CCH, CCH)])

    @pl.when(c == 1)
    def _():
        pltpu.sync_copy(stage_v, o_cd.at[pl.ds(s * CCH, CCH)])


R = 2048


def _rows_spec(width=F):
    return pl.BlockSpec((R, width), lambda i: (i, 0))


def _full_spec(shape):
    return pl.BlockSpec(shape, lambda i: tuple(0 for _ in shape))


def _dot(a, b):
    return jnp.dot(a, b, preferred_element_type=jnp.float32)


def _encode_body(x, wt, b, o):
    o[...] = jnp.maximum(_dot(x[...], wt[...]) + b[...], 0.0)


def _encode(x, enc_wt, enc_b1):
    return pl.pallas_call(
        _encode_body,
        grid=(NP // R,),
        in_specs=[_rows_spec(), _full_spec((F, F)), _full_spec((1, F))],
        out_specs=_rows_spec(),
        out_shape=jax.ShapeDtypeStruct((NP, F), jnp.float32),
    )(x, enc_wt, enc_b1)


def _dense_body(p0, p1, cd1, h, clwt, clb, crwt, glwt, glb, grwt,
                qw1t, qw2t, qb, hn, a_o, b_o, b2_o):
    rcd = 1.0 / jnp.maximum(cd1[...], 1.0)
    agg = (p0[...] + p1[...]) * rcd
    hh = h[...]
    hn[...] = jnp.maximum(_dot(agg, clwt[...]) + clb[...] + _dot(hh, crwt[...]), 0.0)
    xg = jnp.maximum(_dot(agg, glwt[...]) + glb[...] + _dot(hh, grwt[...]), 0.0)
    a_o[...] = _dot(xg, qw1t[...]) + qb[...]
    bb = _dot(xg, qw2t[...])
    b_o[...] = bb
    b2_o[...] = bb * bb


def _dense(p0, p1, cd1, h, clwt, clb, crwt, glwt, glb, grwt, qw1t, qw2t, qb):
    spec = _rows_spec()
    wspec = _full_spec((F, F))
    bspec = _full_spec((1, F))
    return pl.pallas_call(
        _dense_body,
        grid=(NP // R,),
        in_specs=[spec, spec, pl.BlockSpec((R, 1), lambda i: (i, 0)), spec,
                  wspec, bspec, wspec, wspec, bspec, wspec,
                  wspec, wspec, bspec],
        out_specs=[spec, spec, spec, spec],
        out_shape=[jax.ShapeDtypeStruct((NP, F), jnp.float32)] * 4,
    )(p0, p1, cd1, h, clwt, clb, crwt, glwt, glb, grwt, qw1t, qw2t, qb)


def _gate_body(a, s1, s2, cs1, h, hn, o):
    csv = cs1[...]
    rcs = 1.0 / jnp.maximum(csv, 1.0)
    av = a[...]
    m = (csv * av * av + 2.0 * av * s1[...] + s2[...]) * rcs
    tau = jnp.tanh(m)
    o[...] = h[...] + tau * (hn[...] - h[...])


def _gate(a, s1, s2, cs1, h, hn):
    spec = _rows_spec()
    return pl.pallas_call(
        _gate_body,
        grid=(NP // R,),
        in_specs=[spec, spec, spec, pl.BlockSpec((R, 1), lambda i: (i, 0)),
                  spec, spec],
        out_specs=spec,
        out_shape=jax.ShapeDtypeStruct((NP, F), jnp.float32),
    )(a, s1, s2, cs1, h, hn)


def _decode_body(h, wt, b, o):
    o[...] = _dot(h[...], wt[...]) + b[...]


def _decode(h, dec_wt, dec_b1):
    return pl.pallas_call(
        _decode_body,
        grid=(NP // R,),
        in_specs=[_rows_spec(), _full_spec((F, NCLS)), _full_spec((1, NCLS))],
        out_specs=pl.BlockSpec((R, NCLS), lambda i: (i, 0)),
        out_shape=jax.ShapeDtypeStruct((NP, NCLS), jnp.float32),
    )(h, dec_wt, dec_b1)


def kernel(x, edge_index, enc_w, enc_b, dec_w, dec_b, conv_lw, conv_lb,
           conv_rw, gg_lw, gg_lb, gg_rw, q_w, q_b):
    src = edge_index[0]
    dst = edge_index[1]
    xp = jnp.pad(x, ((0, NP - NN), (0, 0)))

    cs, cd = _counts(src, dst)
    cs1 = cs.reshape(NP, 1)
    cd1 = cd.reshape(NP, 1)

    clwt = conv_lw.T
    crwt = conv_rw.T
    glwt = gg_lw.T
    grwt = gg_rw.T
    qw1t = q_w[:, :F].T
    qw2t = q_w[:, F:].T
    clb = conv_lb.reshape(1, F)
    glb = gg_lb.reshape(1, F)
    qb = q_b.reshape(1, F)

    h = _encode(xp, enc_w.T, enc_b.reshape(1, F))
    for _ in range(2):
        p0, p1 = _edge_pass_split(h, h, src, dst)
        hn, a, b, b2 = _dense(p0, p1, cd1, h, clwt, clb, crwt,
                              glwt, glb, grwt, qw1t, qw2t, qb)
        s1, s2 = _edge_pass_dual(b, b2, dst, src)
        h = _gate(a, s1, s2, cs1, h, hn)

    return _decode(h, dec_w.T, dec_b.reshape(1, NCLS))[:NN]
